# baseline (speedup 1.0000x reference)
import functools

import jax
import jax.numpy as jnp
from jax import lax
from jax.experimental import pallas as pl
from jax.experimental.pallas import tpu as pltpu
from jax.experimental.pallas import tpu_sc as plsc

BATCH = 16384
EMB_DIM = 4096
NUM_CORES = 2
NUM_SUBCORES = 16
NUM_WORKERS = NUM_CORES * NUM_SUBCORES
BPW = BATCH // NUM_WORKERS
K = 8
NCHUNK = BPW // K
VECS_PER_ROW = EMB_DIM // 16
UNROLL = 8


def kernel(condition, image_emb, emb_table):
    mesh = plsc.VectorSubcoreMesh(core_axis_name="c", subcore_axis_name="s")

    @functools.partial(
        pl.kernel,
        mesh=mesh,
        out_type=jax.ShapeDtypeStruct((BATCH, EMB_DIM), jnp.float32),
        scratch_types=[
            pltpu.VMEM((BPW,), jnp.int32),
            pltpu.VMEM((K, EMB_DIM), jnp.float32),
            pltpu.VMEM((K, EMB_DIM), jnp.float32),
            pltpu.SemaphoreType.DMA,
            pltpu.SemaphoreType.DMA,
        ],
    )
    def run(cond_hbm, img_hbm, table_hbm, out_hbm, idx_v, rows_v, img_v, sem_g, sem_i):
        wid = lax.axis_index("s") * NUM_CORES + lax.axis_index("c")
        base = wid * BPW
        pltpu.sync_copy(cond_hbm.at[pl.ds(base, BPW)], idx_v)

        def chunk_body(c, carry):
            start = base + c * K
            g = pltpu.async_copy(
                table_hbm.at[idx_v.at[pl.ds(c * K, K)]], rows_v, sem_g
            )
            im = pltpu.async_copy(img_hbm.at[pl.ds(start, K)], img_v, sem_i)
            g.wait()
            im.wait()

            for r in range(K):
                def add_body(j, carry2, r=r):
                    for u in range(UNROLL):
                        sl = pl.ds((j * UNROLL + u) * 16, 16)
                        rows_v[r, sl] = rows_v[r, sl] + img_v[r, sl]
                    return carry2

                lax.fori_loop(0, VECS_PER_ROW // UNROLL, add_body, 0)

            pltpu.sync_copy(rows_v, out_hbm.at[pl.ds(start, K)])
            return carry

        lax.fori_loop(0, NCHUNK, chunk_body, 0)

    return run(condition, image_emb, emb_table)
